# baseline (speedup 1.0000x reference)
import jax
import jax.numpy as jnp
from jax import lax
from jax.experimental import pallas as pl
from jax.experimental.pallas import tpu as pltpu
from jax.experimental.pallas import tpu_sc as plsc

B = 16384
K = 32
NC = 2
NS = 16
NW = NC * NS
BPW = B // NW
CHUNK = 128
NCH = BPW // CHUNK
L = 16


def _body(user_table, item_table, uid, iid, out_hbm,
          idx_u, idx_i, rows_u, rows_i, out_v, sem):
    wid = lax.axis_index("s") * NC + lax.axis_index("c")

    pltpu.sync_copy(uid.at[wid], idx_u)
    pltpu.sync_copy(iid.at[wid], idx_i)

    copies = []
    for j in range(NCH):
        dst = pl.ds(j * CHUNK, CHUNK)
        copies.append(pltpu.async_copy(user_table.at[idx_u.at[j]],
                                       rows_u.at[dst], sem))
        copies.append(pltpu.async_copy(item_table.at[idx_i.at[j]],
                                       rows_i.at[dst], sem))
    for c in copies:
        c.wait()

    def blk(i, _):
        b0 = pl.multiple_of(i * L, L)
        b_idx = b0 + lax.iota(jnp.int32, L)
        acc = jnp.zeros((L,), jnp.float32)
        for k in range(K):
            kv = jnp.full((L,), k, jnp.int32)
            u = plsc.load_gather(rows_u, [b_idx, kv])
            v = plsc.load_gather(rows_i, [b_idx, kv])
            acc = acc + u * v
        out_v[pl.ds(b0, L)] = acc
        return 0

    lax.fori_loop(0, BPW // L, blk, 0)

    pltpu.sync_copy(out_v, out_hbm.at[pl.ds(wid * BPW, BPW)])


def kernel(user_id, item_id, user_table, item_table):
    uid = user_id.astype(jnp.int32).reshape(NW, NCH, CHUNK)
    iid = item_id.astype(jnp.int32).reshape(NW, NCH, CHUNK)
    mesh = plsc.VectorSubcoreMesh(core_axis_name="c", subcore_axis_name="s",
                                  num_cores=NC, num_subcores=NS)
    out = pl.kernel(
        _body,
        out_type=jax.ShapeDtypeStruct((B,), jnp.float32),
        mesh=mesh,
        scratch_types=[
            pltpu.VMEM((NCH, CHUNK), jnp.int32),
            pltpu.VMEM((NCH, CHUNK), jnp.int32),
            pltpu.VMEM((BPW, K), jnp.float32),
            pltpu.VMEM((BPW, K), jnp.float32),
            pltpu.VMEM((BPW,), jnp.float32),
            pltpu.SemaphoreType.DMA,
        ],
        compiler_params=pltpu.CompilerParams(needs_layout_passes=False,
                                             use_tc_tiling_on_sc=False),
    )(user_table, item_table, uid, iid)
    return out.reshape(B, 1)
